# baseline (device time: 356133 ns/iter reference)
import jax
import jax.numpy as jnp
from jax import lax
from jax.experimental import pallas as pl
from jax.experimental.pallas import tpu as pltpu

N_DEV = 8


def kernel(x, w_mat):
    m, k_shard = x.shape
    _, n = w_mat.shape
    chunk = m // N_DEV

    def body(x_ref, w_ref, out_ref, recv_ref,
             rs_send_sems, rs_recv_sems, ag_send_sems, ag_recv_sems):
        my = lax.axis_index("i")
        left = lax.rem(my + (N_DEV - 1), N_DEV)
        right = lax.rem(my + 1, N_DEV)

        barrier_sem = pltpu.get_barrier_semaphore()
        for nbr in (left, right):
            pl.semaphore_signal(
                barrier_sem, inc=1,
                device_id=(nbr,), device_id_type=pl.DeviceIdType.MESH,
            )
        pl.semaphore_wait(barrier_sem, 2)

        out_ref[...] = jnp.dot(
            x_ref[...], w_ref[...], preferred_element_type=jnp.float32
        )

        for s in range(N_DEV - 1):
            send_idx = lax.rem(my + (N_DEV - s), N_DEV)
            add_idx = lax.rem(my + (N_DEV - 1 - s), N_DEV)
            rdma = pltpu.make_async_remote_copy(
                src_ref=out_ref.at[pl.ds(send_idx * chunk, chunk), :],
                dst_ref=recv_ref.at[s],
                send_sem=rs_send_sems.at[s],
                recv_sem=rs_recv_sems.at[s],
                device_id=(right,),
                device_id_type=pl.DeviceIdType.MESH,
            )
            rdma.start()
            rdma.wait()
            out_ref[pl.ds(add_idx * chunk, chunk), :] += recv_ref[s]

        own = lax.rem(my + 1, N_DEV)
        out_ref[pl.ds(own * chunk, chunk), :] = jnp.maximum(
            out_ref[pl.ds(own * chunk, chunk), :], 0.0
        )

        for s in range(N_DEV - 1):
            send_idx = lax.rem(my + 1 + (N_DEV - s), N_DEV)
            rdma = pltpu.make_async_remote_copy(
                src_ref=out_ref.at[pl.ds(send_idx * chunk, chunk), :],
                dst_ref=out_ref.at[pl.ds(send_idx * chunk, chunk), :],
                send_sem=ag_send_sems.at[s],
                recv_sem=ag_recv_sems.at[s],
                device_id=(right,),
                device_id_type=pl.DeviceIdType.MESH,
            )
            rdma.start()
            rdma.wait()

    return pl.pallas_call(
        body,
        out_shape=jax.ShapeDtypeStruct((m, n), jnp.float32),
        in_specs=[
            pl.BlockSpec(memory_space=pltpu.VMEM),
            pl.BlockSpec(memory_space=pltpu.VMEM),
        ],
        out_specs=pl.BlockSpec(memory_space=pltpu.VMEM),
        scratch_shapes=[
            pltpu.VMEM((N_DEV - 1, chunk, n), jnp.float32),
            pltpu.SemaphoreType.DMA((N_DEV - 1,)),
            pltpu.SemaphoreType.DMA((N_DEV - 1,)),
            pltpu.SemaphoreType.DMA((N_DEV - 1,)),
            pltpu.SemaphoreType.DMA((N_DEV - 1,)),
        ],
        compiler_params=pltpu.CompilerParams(collective_id=0),
    )(x, w_mat)


# device time: 203255 ns/iter; 1.7521x vs baseline; 1.7521x over previous
import jax
import jax.numpy as jnp
from jax import lax
from jax.experimental import pallas as pl
from jax.experimental.pallas import tpu as pltpu

N_DEV = 8


def kernel(x, w_mat):
    m, k_shard = x.shape
    _, n = w_mat.shape
    half = m // 2
    chunk = half // N_DEV

    def body(x_ref, w_ref, out_ref, recv_f, recv_b,
             sems_f, sems_b, ag_sems_f, ag_sems_b):
        my = lax.axis_index("i")
        left = lax.rem(my + (N_DEV - 1), N_DEV)
        right = lax.rem(my + 1, N_DEV)

        barrier_sem = pltpu.get_barrier_semaphore()
        for nbr in (left, right):
            pl.semaphore_signal(
                barrier_sem, inc=1,
                device_id=(nbr,), device_id_type=pl.DeviceIdType.MESH,
            )
        pl.semaphore_wait(barrier_sem, 2)

        out_ref[...] = jnp.dot(
            x_ref[...], w_ref[...], preferred_element_type=jnp.float32
        )

        def f_rows(c):
            return pl.ds(c * chunk, chunk)

        def b_rows(c):
            return pl.ds(half + c * chunk, chunk)

        for s in range(N_DEV - 1):
            sf = lax.rem(my + (N_DEV - s), N_DEV)
            af = lax.rem(my + (N_DEV - 1 - s), N_DEV)
            sb = lax.rem(my + s, N_DEV)
            ab = lax.rem(my + s + 1, N_DEV)
            rdma_f = pltpu.make_async_remote_copy(
                src_ref=out_ref.at[f_rows(sf), :],
                dst_ref=recv_f.at[s],
                send_sem=sems_f.at[0, s],
                recv_sem=sems_f.at[1, s],
                device_id=(right,),
                device_id_type=pl.DeviceIdType.MESH,
            )
            rdma_b = pltpu.make_async_remote_copy(
                src_ref=out_ref.at[b_rows(sb), :],
                dst_ref=recv_b.at[s],
                send_sem=sems_b.at[0, s],
                recv_sem=sems_b.at[1, s],
                device_id=(left,),
                device_id_type=pl.DeviceIdType.MESH,
            )
            rdma_f.start()
            rdma_b.start()
            rdma_f.wait()
            out_ref[f_rows(af), :] += recv_f[s]
            rdma_b.wait()
            out_ref[b_rows(ab), :] += recv_b[s]

        own_f = lax.rem(my + 1, N_DEV)
        own_b = left
        out_ref[f_rows(own_f), :] = jnp.maximum(out_ref[f_rows(own_f), :], 0.0)
        out_ref[b_rows(own_b), :] = jnp.maximum(out_ref[b_rows(own_b), :], 0.0)

        for s in range(N_DEV - 1):
            sf = lax.rem(my + 1 + (N_DEV - s), N_DEV)
            sb = lax.rem(my + (N_DEV - 1) + s, N_DEV)
            rdma_f = pltpu.make_async_remote_copy(
                src_ref=out_ref.at[f_rows(sf), :],
                dst_ref=out_ref.at[f_rows(sf), :],
                send_sem=ag_sems_f.at[0, s],
                recv_sem=ag_sems_f.at[1, s],
                device_id=(right,),
                device_id_type=pl.DeviceIdType.MESH,
            )
            rdma_b = pltpu.make_async_remote_copy(
                src_ref=out_ref.at[b_rows(sb), :],
                dst_ref=out_ref.at[b_rows(sb), :],
                send_sem=ag_sems_b.at[0, s],
                recv_sem=ag_sems_b.at[1, s],
                device_id=(left,),
                device_id_type=pl.DeviceIdType.MESH,
            )
            rdma_f.start()
            rdma_b.start()
            rdma_f.wait()
            rdma_b.wait()

    return pl.pallas_call(
        body,
        out_shape=jax.ShapeDtypeStruct((m, n), jnp.float32),
        in_specs=[
            pl.BlockSpec(memory_space=pltpu.VMEM),
            pl.BlockSpec(memory_space=pltpu.VMEM),
        ],
        out_specs=pl.BlockSpec(memory_space=pltpu.VMEM),
        scratch_shapes=[
            pltpu.VMEM((N_DEV - 1, chunk, n), jnp.float32),
            pltpu.VMEM((N_DEV - 1, chunk, n), jnp.float32),
            pltpu.SemaphoreType.DMA((2, N_DEV - 1)),
            pltpu.SemaphoreType.DMA((2, N_DEV - 1)),
            pltpu.SemaphoreType.DMA((2, N_DEV - 1)),
            pltpu.SemaphoreType.DMA((2, N_DEV - 1)),
        ],
        compiler_params=pltpu.CompilerParams(collective_id=0),
    )(x, w_mat)


# device time: 135096 ns/iter; 2.6361x vs baseline; 1.5045x over previous
import jax
import jax.numpy as jnp
from jax import lax
from jax.experimental import pallas as pl
from jax.experimental.pallas import tpu as pltpu

N_DEV = 8
PARTS = (640, 704, 704)
DIM_XOR = (1, 3, 4)


def kernel(x, w_mat):
    m, _ = x.shape
    _, n = w_mat.shape
    assert sum(PARTS) == m

    part_starts = []
    recv_bases = []
    off = 0
    roff = 0
    for p_len in PARTS:
        part_starts.append(off)
        off += p_len
        recv_bases.append((roff, (0, p_len // 2, p_len // 2 + p_len // 4)))
        roff += p_len // 2 + p_len // 4 + p_len // 8
    recv_rows = roff

    def body(x_ref, w_ref, out_ref, recv_ref, rs_sems, ag_sems):
        my = lax.axis_index("i")
        my4 = lax.rem(my, 4)
        bit = [
            jnp.where((my4 == 1) | (my4 == 2), 1, 0),
            jnp.where(my4 >= 2, 1, 0),
            jnp.where(my >= 4, 1, 0),
        ]
        partner = [jnp.bitwise_xor(my, DIM_XOR[d]) for d in range(3)]

        barrier_sem = pltpu.get_barrier_semaphore()
        for d in range(3):
            pl.semaphore_signal(
                barrier_sem, inc=1,
                device_id=(partner[d],), device_id_type=pl.DeviceIdType.MESH,
            )
        pl.semaphore_wait(barrier_sem, 3)

        out_ref[...] = jnp.dot(
            x_ref[...], w_ref[...], preferred_element_type=jnp.float32
        )

        o = [part_starts[p] + jnp.int32(0) for p in range(3)]
        L = [PARTS[p] for p in range(3)]

        for k in range(3):
            started = []
            for p in range(3):
                d = (p + k) % 3
                half = L[p] // 2
                send_off = o[p] + (1 - bit[d]) * half
                base, slots = recv_bases[p]
                slot_off = base + slots[k]
                rdma = pltpu.make_async_remote_copy(
                    src_ref=out_ref.at[pl.ds(send_off, half), :],
                    dst_ref=recv_ref.at[pl.ds(slot_off, half), :],
                    send_sem=rs_sems.at[p, 0, k],
                    recv_sem=rs_sems.at[p, 1, k],
                    device_id=(partner[d],),
                    device_id_type=pl.DeviceIdType.MESH,
                )
                rdma.start()
                started.append((p, d, half, slot_off, rdma))
            for p, d, half, slot_off, rdma in started:
                rdma.wait()
                keep = o[p] + bit[d] * half
                out_ref[pl.ds(keep, half), :] += (
                    recv_ref[pl.ds(slot_off, half), :]
                )
                o[p] = keep
                L[p] = half

        for p in range(3):
            out_ref[pl.ds(o[p], L[p]), :] = jnp.maximum(
                out_ref[pl.ds(o[p], L[p]), :], 0.0
            )

        for k in range(3):
            started = []
            for p in range(3):
                d = (p + 2 - k) % 3
                rdma = pltpu.make_async_remote_copy(
                    src_ref=out_ref.at[pl.ds(o[p], L[p]), :],
                    dst_ref=out_ref.at[pl.ds(o[p], L[p]), :],
                    send_sem=ag_sems.at[p, 0, k],
                    recv_sem=ag_sems.at[p, 1, k],
                    device_id=(partner[d],),
                    device_id_type=pl.DeviceIdType.MESH,
                )
                rdma.start()
                started.append((p, d, rdma))
            for p, d, rdma in started:
                rdma.wait()
                o[p] = o[p] - bit[d] * L[p]
                L[p] = L[p] * 2

    return pl.pallas_call(
        body,
        out_shape=jax.ShapeDtypeStruct((m, n), jnp.float32),
        in_specs=[
            pl.BlockSpec(memory_space=pltpu.VMEM),
            pl.BlockSpec(memory_space=pltpu.VMEM),
        ],
        out_specs=pl.BlockSpec(memory_space=pltpu.VMEM),
        scratch_shapes=[
            pltpu.VMEM((recv_rows, n), jnp.float32),
            pltpu.SemaphoreType.DMA((3, 2, 3)),
            pltpu.SemaphoreType.DMA((3, 2, 3)),
        ],
        compiler_params=pltpu.CompilerParams(collective_id=0),
    )(x, w_mat)


# device time: 134906 ns/iter; 2.6399x vs baseline; 1.0014x over previous
import jax
import jax.numpy as jnp
from jax import lax
from jax.experimental import pallas as pl
from jax.experimental.pallas import tpu as pltpu

N_DEV = 8
PARTS = (640, 704, 704)
DIM_XOR = (1, 3, 4)


def kernel(x, w_mat):
    m, _ = x.shape
    _, n = w_mat.shape
    assert sum(PARTS) == m

    part_starts = []
    recv_bases = []
    off = 0
    roff = 0
    for p_len in PARTS:
        part_starts.append(off)
        off += p_len
        recv_bases.append((roff, (0, p_len // 2, p_len // 2 + p_len // 4)))
        roff += p_len // 2 + p_len // 4 + p_len // 8
    recv_rows = roff

    def body(x_ref, w_ref, out_ref, recv_ref, rs_sems, ag_sems):
        my = lax.axis_index("i")
        my4 = lax.rem(my, 4)
        bit = [
            jnp.where((my4 == 1) | (my4 == 2), 1, 0),
            jnp.where(my4 >= 2, 1, 0),
            jnp.where(my >= 4, 1, 0),
        ]
        partner = [jnp.bitwise_xor(my, DIM_XOR[d]) for d in range(3)]

        barrier_sem = pltpu.get_barrier_semaphore()
        for d in range(3):
            pl.semaphore_signal(
                barrier_sem, inc=1,
                device_id=(partner[d],), device_id_type=pl.DeviceIdType.MESH,
            )
        pl.semaphore_wait(barrier_sem, 3)

        out_ref[...] = jnp.dot(
            x_ref[...], w_ref[...], preferred_element_type=jnp.float32
        )

        o = [part_starts[p] + jnp.int32(0) for p in range(3)]
        L = [PARTS[p] for p in range(3)]

        for k in range(2):
            started = []
            for p in range(3):
                d = (p + k) % 3
                half = L[p] // 2
                sub = half // 2
                send_off = o[p] + (1 - bit[d]) * half
                base, slots = recv_bases[p]
                slot_off = base + slots[k]
                rdmas = []
                for j in range(2):
                    r = pltpu.make_async_remote_copy(
                        src_ref=out_ref.at[pl.ds(send_off + j * sub, sub), :],
                        dst_ref=recv_ref.at[pl.ds(slot_off + j * sub, sub), :],
                        send_sem=rs_sems.at[p, 0, k, j],
                        recv_sem=rs_sems.at[p, 1, k, j],
                        device_id=(partner[d],),
                        device_id_type=pl.DeviceIdType.MESH,
                    )
                    r.start()
                    rdmas.append(r)
                started.append((p, d, half, sub, slot_off, rdmas))
            for p, d, half, sub, slot_off, rdmas in started:
                keep = o[p] + bit[d] * half
                for j in range(2):
                    rdmas[j].wait()
                    out_ref[pl.ds(keep + j * sub, sub), :] += (
                        recv_ref[pl.ds(slot_off + j * sub, sub), :]
                    )
                o[p] = keep
                L[p] = half

        started = []
        for p in range(3):
            d = (p + 2) % 3
            half = L[p] // 2
            send_off = o[p] + (1 - bit[d]) * half
            base, slots = recv_bases[p]
            slot_off = base + slots[2]
            r = pltpu.make_async_remote_copy(
                src_ref=out_ref.at[pl.ds(send_off, half), :],
                dst_ref=recv_ref.at[pl.ds(slot_off, half), :],
                send_sem=rs_sems.at[p, 0, 2, 0],
                recv_sem=rs_sems.at[p, 1, 2, 0],
                device_id=(partner[d],),
                device_id_type=pl.DeviceIdType.MESH,
            )
            r.start()
            started.append((p, d, half, slot_off, r))
        ag_started = []
        for p, d, half, slot_off, r in started:
            r.wait()
            keep = o[p] + bit[d] * half
            out_ref[pl.ds(keep, half), :] = jnp.maximum(
                out_ref[pl.ds(keep, half), :]
                + recv_ref[pl.ds(slot_off, half), :],
                0.0,
            )
            o[p] = keep
            L[p] = half
            ag0 = pltpu.make_async_remote_copy(
                src_ref=out_ref.at[pl.ds(o[p], L[p]), :],
                dst_ref=out_ref.at[pl.ds(o[p], L[p]), :],
                send_sem=ag_sems.at[p, 0, 0],
                recv_sem=ag_sems.at[p, 1, 0],
                device_id=(partner[d],),
                device_id_type=pl.DeviceIdType.MESH,
            )
            ag0.start()
            ag_started.append((p, d, ag0))

        for p, d, ag0 in ag_started:
            ag0.wait()
            o[p] = o[p] - bit[d] * L[p]
            L[p] = L[p] * 2

        for k in range(1, 3):
            started = []
            for p in range(3):
                d = (p + 2 - k) % 3
                rdma = pltpu.make_async_remote_copy(
                    src_ref=out_ref.at[pl.ds(o[p], L[p]), :],
                    dst_ref=out_ref.at[pl.ds(o[p], L[p]), :],
                    send_sem=ag_sems.at[p, 0, k],
                    recv_sem=ag_sems.at[p, 1, k],
                    device_id=(partner[d],),
                    device_id_type=pl.DeviceIdType.MESH,
                )
                rdma.start()
                started.append((p, d, rdma))
            for p, d, rdma in started:
                rdma.wait()
                o[p] = o[p] - bit[d] * L[p]
                L[p] = L[p] * 2

    return pl.pallas_call(
        body,
        out_shape=jax.ShapeDtypeStruct((m, n), jnp.float32),
        in_specs=[
            pl.BlockSpec(memory_space=pltpu.VMEM),
            pl.BlockSpec(memory_space=pltpu.VMEM),
        ],
        out_specs=pl.BlockSpec(memory_space=pltpu.VMEM),
        scratch_shapes=[
            pltpu.VMEM((recv_rows, n), jnp.float32),
            pltpu.SemaphoreType.DMA((3, 2, 3, 2)),
            pltpu.SemaphoreType.DMA((3, 2, 3)),
        ],
        compiler_params=pltpu.CompilerParams(collective_id=0),
    )(x, w_mat)


# device time: 133866 ns/iter; 2.6604x vs baseline; 1.0078x over previous
import contextlib
import os

import jax
import jax.numpy as jnp
from jax import lax
from jax.experimental import pallas as pl
from jax.experimental.pallas import tpu as pltpu

_PROF = os.environ.get("KPROF") == "1"
_PHASE = os.environ.get("KPHASE", "full")


def _scope(name):
    return jax.named_scope(name) if _PROF else contextlib.nullcontext()


N_DEV = 8
PARTS = (640, 704, 704)
DIM_XOR = (1, 3, 4)


def kernel(x, w_mat):
    m, _ = x.shape
    _, n = w_mat.shape
    assert sum(PARTS) == m

    part_starts = []
    recv_bases = []
    off = 0
    roff = 0
    for p_len in PARTS:
        part_starts.append(off)
        off += p_len
        recv_bases.append((roff, (0, p_len // 2, p_len // 2 + p_len // 4)))
        roff += p_len // 2 + p_len // 4 + p_len // 8
    recv_rows = roff

    def body(x_ref, w_ref, out_ref, recv_ref, rs_sems, ag_sems):
        my = lax.axis_index("i")
        my4 = lax.rem(my, 4)
        bit = [
            jnp.where((my4 == 1) | (my4 == 2), 1, 0),
            jnp.where(my4 >= 2, 1, 0),
            jnp.where(my >= 4, 1, 0),
        ]
        partner = [jnp.bitwise_xor(my, DIM_XOR[d]) for d in range(3)]

        with _scope("barrier"):
            barrier_sem = pltpu.get_barrier_semaphore()
            for d in range(3):
                pl.semaphore_signal(
                    barrier_sem, inc=1,
                    device_id=(partner[d],),
                    device_id_type=pl.DeviceIdType.MESH,
                )
            pl.semaphore_wait(barrier_sem, 3)

        if _PHASE == "gemm":
            with _scope("gemm"):
                out_ref[...] = jnp.dot(
                    x_ref[...], w_ref[...], preferred_element_type=jnp.float32
                )
            return

        o = [part_starts[p] + jnp.int32(0) for p in range(3)]
        L = [PARTS[p] for p in range(3)]

        with _scope("rs0_gemm"):
            started = []
            for p in range(3):
                d = p
                half = L[p] // 2
                sub = half // 2
                send_off = o[p] + (1 - bit[d]) * half
                out_ref[pl.ds(send_off, half), :] = jnp.dot(
                    x_ref[pl.ds(send_off, half), :], w_ref[...],
                    preferred_element_type=jnp.float32,
                )
                base, slots = recv_bases[p]
                slot_off = base + slots[0]
                rdmas = []
                for j in range(2):
                    r = pltpu.make_async_remote_copy(
                        src_ref=out_ref.at[pl.ds(send_off + j * sub, sub), :],
                        dst_ref=recv_ref.at[pl.ds(slot_off + j * sub, sub), :],
                        send_sem=rs_sems.at[p, 0, 0, j],
                        recv_sem=rs_sems.at[p, 1, 0, j],
                        device_id=(partner[d],),
                        device_id_type=pl.DeviceIdType.MESH,
                    )
                    r.start()
                    rdmas.append(r)
                started.append((p, d, half, sub, slot_off, rdmas))
            for p in range(3):
                half = L[p] // 2
                keep_off = o[p] + bit[p] * half
                out_ref[pl.ds(keep_off, half), :] = jnp.dot(
                    x_ref[pl.ds(keep_off, half), :], w_ref[...],
                    preferred_element_type=jnp.float32,
                )
            for p, d, half, sub, slot_off, rdmas in started:
                keep = o[p] + bit[d] * half
                for j in range(2):
                    rdmas[j].wait()
                    out_ref[pl.ds(keep + j * sub, sub), :] += (
                        recv_ref[pl.ds(slot_off + j * sub, sub), :]
                    )
                o[p] = keep
                L[p] = half

        for k in range(1, 2):
            with _scope(f"rs{k}"):
                started = []
                for p in range(3):
                    d = (p + k) % 3
                    half = L[p] // 2
                    sub = half // 2
                    send_off = o[p] + (1 - bit[d]) * half
                    base, slots = recv_bases[p]
                    slot_off = base + slots[k]
                    rdmas = []
                    for j in range(2):
                        r = pltpu.make_async_remote_copy(
                            src_ref=out_ref.at[
                                pl.ds(send_off + j * sub, sub), :
                            ],
                            dst_ref=recv_ref.at[
                                pl.ds(slot_off + j * sub, sub), :
                            ],
                            send_sem=rs_sems.at[p, 0, k, j],
                            recv_sem=rs_sems.at[p, 1, k, j],
                            device_id=(partner[d],),
                            device_id_type=pl.DeviceIdType.MESH,
                        )
                        r.start()
                        rdmas.append(r)
                    started.append((p, d, half, sub, slot_off, rdmas))
                for p, d, half, sub, slot_off, rdmas in started:
                    keep = o[p] + bit[d] * half
                    for j in range(2):
                        rdmas[j].wait()
                        out_ref[pl.ds(keep + j * sub, sub), :] += (
                            recv_ref[pl.ds(slot_off + j * sub, sub), :]
                        )
                    o[p] = keep
                    L[p] = half

        with _scope("rs2_ag0start"):
            started = []
            for p in range(3):
                d = (p + 2) % 3
                half = L[p] // 2
                send_off = o[p] + (1 - bit[d]) * half
                base, slots = recv_bases[p]
                slot_off = base + slots[2]
                r = pltpu.make_async_remote_copy(
                    src_ref=out_ref.at[pl.ds(send_off, half), :],
                    dst_ref=recv_ref.at[pl.ds(slot_off, half), :],
                    send_sem=rs_sems.at[p, 0, 2, 0],
                    recv_sem=rs_sems.at[p, 1, 2, 0],
                    device_id=(partner[d],),
                    device_id_type=pl.DeviceIdType.MESH,
                )
                r.start()
                started.append((p, d, half, slot_off, r))
            ag_started = []
            for p, d, half, slot_off, r in started:
                r.wait()
                keep = o[p] + bit[d] * half
                out_ref[pl.ds(keep, half), :] = jnp.maximum(
                    out_ref[pl.ds(keep, half), :]
                    + recv_ref[pl.ds(slot_off, half), :],
                    0.0,
                )
                o[p] = keep
                L[p] = half
                if _PHASE == "full":
                    ag0 = pltpu.make_async_remote_copy(
                        src_ref=out_ref.at[pl.ds(o[p], L[p]), :],
                        dst_ref=out_ref.at[pl.ds(o[p], L[p]), :],
                        send_sem=ag_sems.at[p, 0, 0],
                        recv_sem=ag_sems.at[p, 1, 0],
                        device_id=(partner[d],),
                        device_id_type=pl.DeviceIdType.MESH,
                    )
                    ag0.start()
                    ag_started.append((p, d, ag0))

        if _PHASE == "rs":
            return

        with _scope("ag0_wait"):
            for p, d, ag0 in ag_started:
                ag0.wait()
                o[p] = o[p] - bit[d] * L[p]
                L[p] = L[p] * 2

        for k in range(1, 3):
            with _scope(f"ag{k}"):
                started = []
                for p in range(3):
                    d = (p + 2 - k) % 3
                    rdma = pltpu.make_async_remote_copy(
                        src_ref=out_ref.at[pl.ds(o[p], L[p]), :],
                        dst_ref=out_ref.at[pl.ds(o[p], L[p]), :],
                        send_sem=ag_sems.at[p, 0, k],
                        recv_sem=ag_sems.at[p, 1, k],
                        device_id=(partner[d],),
                        device_id_type=pl.DeviceIdType.MESH,
                    )
                    rdma.start()
                    started.append((p, d, rdma))
                for p, d, rdma in started:
                    rdma.wait()
                    o[p] = o[p] - bit[d] * L[p]
                    L[p] = L[p] * 2

    return pl.pallas_call(
        body,
        out_shape=jax.ShapeDtypeStruct((m, n), jnp.float32),
        in_specs=[
            pl.BlockSpec(memory_space=pltpu.VMEM),
            pl.BlockSpec(memory_space=pltpu.VMEM),
        ],
        out_specs=pl.BlockSpec(memory_space=pltpu.VMEM),
        scratch_shapes=[
            pltpu.VMEM((recv_rows, n), jnp.float32),
            pltpu.SemaphoreType.DMA((3, 2, 3, 2)),
            pltpu.SemaphoreType.DMA((3, 2, 3)),
        ],
        compiler_params=pltpu.CompilerParams(collective_id=0),
    )(x, w_mat)


# device time: 133818 ns/iter; 2.6613x vs baseline; 1.0004x over previous
import contextlib
import os

import jax
import jax.numpy as jnp
from jax import lax
from jax.experimental import pallas as pl
from jax.experimental.pallas import tpu as pltpu

_PROF = os.environ.get("KPROF") == "1"
_PHASE = os.environ.get("KPHASE", "full")


def _scope(name):
    return jax.named_scope(name) if _PROF else contextlib.nullcontext()


N_DEV = 8
PARTS = (640, 704, 704)
DIM_XOR = (1, 3, 4)


def kernel(x, w_mat):
    m, _ = x.shape
    _, n = w_mat.shape
    assert sum(PARTS) == m

    part_starts = []
    recv_bases = []
    off = 0
    roff = 0
    for p_len in PARTS:
        part_starts.append(off)
        off += p_len
        recv_bases.append((roff, (0, p_len // 2, p_len // 2 + p_len // 4)))
        roff += p_len // 2 + p_len // 4 + p_len // 8
    recv_rows = roff

    sb_bases = []
    sb_off = 0
    for p_len in PARTS:
        sb_bases.append(sb_off)
        sb_off += p_len // 2
    sb_rows = sb_off

    def body(x_ref, w_ref, out_ref, recv_ref, send_buf, rs_sems, ag_sems):
        my = lax.axis_index("i")
        my4 = lax.rem(my, 4)
        bit = [
            jnp.where((my4 == 1) | (my4 == 2), 1, 0),
            jnp.where(my4 >= 2, 1, 0),
            jnp.where(my >= 4, 1, 0),
        ]
        partner = [jnp.bitwise_xor(my, DIM_XOR[d]) for d in range(3)]

        with _scope("barrier"):
            barrier_sem = pltpu.get_barrier_semaphore()
            for d in range(3):
                pl.semaphore_signal(
                    barrier_sem, inc=1,
                    device_id=(partner[d],),
                    device_id_type=pl.DeviceIdType.MESH,
                )
            pl.semaphore_wait(barrier_sem, 3)

        if _PHASE == "gemm":
            with _scope("gemm"):
                out_ref[...] = jnp.dot(
                    x_ref[...], w_ref[...], preferred_element_type=jnp.float32
                )
            return

        if _PHASE == "gemm6":
            for p in range(3):
                half = PARTS[p] // 2
                send_off = part_starts[p] + (1 - bit[p]) * half
                out_ref[pl.ds(send_off, half), :] = jnp.dot(
                    x_ref[pl.ds(send_off, half), :], w_ref[...],
                    preferred_element_type=jnp.float32,
                )
            for p in range(3):
                half = PARTS[p] // 2
                keep_off = part_starts[p] + bit[p] * half
                out_ref[pl.ds(keep_off, half), :] = jnp.dot(
                    x_ref[pl.ds(keep_off, half), :], w_ref[...],
                    preferred_element_type=jnp.float32,
                )
            return

        o = [part_starts[p] + jnp.int32(0) for p in range(3)]
        L = [PARTS[p] for p in range(3)]

        with _scope("rs0_gemm"):
            started = []
            for p in range(3):
                d = p
                half = L[p] // 2
                sub = half // 2
                send_off = o[p] + (1 - bit[d]) * half
                sb = sb_bases[p]
                send_buf[pl.ds(sb, half), :] = jnp.dot(
                    x_ref[pl.ds(send_off, half), :], w_ref[...],
                    preferred_element_type=jnp.float32,
                )
                base, slots = recv_bases[p]
                slot_off = base + slots[0]
                rdmas = []
                for j in range(2):
                    r = pltpu.make_async_remote_copy(
                        src_ref=send_buf.at[pl.ds(sb + j * sub, sub), :],
                        dst_ref=recv_ref.at[pl.ds(slot_off + j * sub, sub), :],
                        send_sem=rs_sems.at[p, 0, 0, j],
                        recv_sem=rs_sems.at[p, 1, 0, j],
                        device_id=(partner[d],),
                        device_id_type=pl.DeviceIdType.MESH,
                    )
                    r.start()
                    rdmas.append(r)
                started.append((p, d, half, sub, slot_off, rdmas))
            for p in range(3):
                half = L[p] // 2
                keep_off = o[p] + bit[p] * half
                out_ref[pl.ds(keep_off, half), :] = jnp.dot(
                    x_ref[pl.ds(keep_off, half), :], w_ref[...],
                    preferred_element_type=jnp.float32,
                )
            for p, d, half, sub, slot_off, rdmas in started:
                keep = o[p] + bit[d] * half
                for j in range(2):
                    rdmas[j].wait()
                    out_ref[pl.ds(keep + j * sub, sub), :] += (
                        recv_ref[pl.ds(slot_off + j * sub, sub), :]
                    )
                o[p] = keep
                L[p] = half

        for k in range(1, 2):
            with _scope(f"rs{k}"):
                started = []
                for p in range(3):
                    d = (p + k) % 3
                    half = L[p] // 2
                    sub = half // 2
                    send_off = o[p] + (1 - bit[d]) * half
                    base, slots = recv_bases[p]
                    slot_off = base + slots[k]
                    rdmas = []
                    for j in range(2):
                        r = pltpu.make_async_remote_copy(
                            src_ref=out_ref.at[
                                pl.ds(send_off + j * sub, sub), :
                            ],
                            dst_ref=recv_ref.at[
                                pl.ds(slot_off + j * sub, sub), :
                            ],
                            send_sem=rs_sems.at[p, 0, k, j],
                            recv_sem=rs_sems.at[p, 1, k, j],
                            device_id=(partner[d],),
                            device_id_type=pl.DeviceIdType.MESH,
                        )
                        r.start()
                        rdmas.append(r)
                    started.append((p, d, half, sub, slot_off, rdmas))
                for p, d, half, sub, slot_off, rdmas in started:
                    keep = o[p] + bit[d] * half
                    for j in range(2):
                        rdmas[j].wait()
                        out_ref[pl.ds(keep + j * sub, sub), :] += (
                            recv_ref[pl.ds(slot_off + j * sub, sub), :]
                        )
                    o[p] = keep
                    L[p] = half

        with _scope("rs2_ag0start"):
            started = []
            for p in range(3):
                d = (p + 2) % 3
                half = L[p] // 2
                send_off = o[p] + (1 - bit[d]) * half
                base, slots = recv_bases[p]
                slot_off = base + slots[2]
                r = pltpu.make_async_remote_copy(
                    src_ref=out_ref.at[pl.ds(send_off, half), :],
                    dst_ref=recv_ref.at[pl.ds(slot_off, half), :],
                    send_sem=rs_sems.at[p, 0, 2, 0],
                    recv_sem=rs_sems.at[p, 1, 2, 0],
                    device_id=(partner[d],),
                    device_id_type=pl.DeviceIdType.MESH,
                )
                r.start()
                started.append((p, d, half, slot_off, r))
            ag_started = []
            for p, d, half, slot_off, r in started:
                r.wait()
                keep = o[p] + bit[d] * half
                out_ref[pl.ds(keep, half), :] = jnp.maximum(
                    out_ref[pl.ds(keep, half), :]
                    + recv_ref[pl.ds(slot_off, half), :],
                    0.0,
                )
                o[p] = keep
                L[p] = half
                if _PHASE == "full":
                    ag0 = pltpu.make_async_remote_copy(
                        src_ref=out_ref.at[pl.ds(o[p], L[p]), :],
                        dst_ref=out_ref.at[pl.ds(o[p], L[p]), :],
                        send_sem=ag_sems.at[p, 0, 0],
                        recv_sem=ag_sems.at[p, 1, 0],
                        device_id=(partner[d],),
                        device_id_type=pl.DeviceIdType.MESH,
                    )
                    ag0.start()
                    ag_started.append((p, d, ag0))

        if _PHASE == "rs":
            return

        with _scope("ag0_wait"):
            for p, d, ag0 in ag_started:
                ag0.wait()
                o[p] = o[p] - bit[d] * L[p]
                L[p] = L[p] * 2

        for k in range(1, 3):
            with _scope(f"ag{k}"):
                started = []
                for p in range(3):
                    d = (p + 2 - k) % 3
                    rdma = pltpu.make_async_remote_copy(
                        src_ref=out_ref.at[pl.ds(o[p], L[p]), :],
                        dst_ref=out_ref.at[pl.ds(o[p], L[p]), :],
                        send_sem=ag_sems.at[p, 0, k],
                        recv_sem=ag_sems.at[p, 1, k],
                        device_id=(partner[d],),
                        device_id_type=pl.DeviceIdType.MESH,
                    )
                    rdma.start()
                    started.append((p, d, rdma))
                for p, d, rdma in started:
                    rdma.wait()
                    o[p] = o[p] - bit[d] * L[p]
                    L[p] = L[p] * 2

    return pl.pallas_call(
        body,
        out_shape=jax.ShapeDtypeStruct((m, n), jnp.float32),
        in_specs=[
            pl.BlockSpec(memory_space=pltpu.VMEM),
            pl.BlockSpec(memory_space=pltpu.VMEM),
        ],
        out_specs=pl.BlockSpec(memory_space=pltpu.VMEM),
        scratch_shapes=[
            pltpu.VMEM((recv_rows, n), jnp.float32),
            pltpu.VMEM((sb_rows, n), jnp.float32),
            pltpu.SemaphoreType.DMA((3, 2, 3, 2)),
            pltpu.SemaphoreType.DMA((3, 2, 3)),
        ],
        compiler_params=pltpu.CompilerParams(collective_id=0),
    )(x, w_mat)


# device time: 84689 ns/iter; 4.2052x vs baseline; 1.5801x over previous
import jax
import jax.numpy as jnp
from jax import lax
from jax.experimental import pallas as pl
from jax.experimental.pallas import tpu as pltpu

N_DEV = 8
PARTS = (640, 640, 768)
DIM_XOR = (1, 3, 4)


def kernel(x, w_mat):
    m, _ = x.shape
    _, n = w_mat.shape
    assert sum(PARTS) == m

    part_starts = []
    comm_bases = []
    off = 0
    roff = 0
    for p_len in PARTS:
        part_starts.append(off)
        off += p_len
        comm_bases.append((roff, (0, p_len // 2, p_len // 2 + p_len // 4)))
        roff += p_len // 2 + p_len // 4 + p_len // 8
    comm_rows = roff

    f32 = jnp.float32
    bf16 = jnp.bfloat16

    def body(x_ref, w_ref, out_ref, recv_bf, send_bf, ag_bf,
             rs_sems, ag_sems):
        my = lax.axis_index("i")
        my4 = lax.rem(my, 4)
        bit = [
            jnp.where((my4 == 1) | (my4 == 2), 1, 0),
            jnp.where(my4 >= 2, 1, 0),
            jnp.where(my >= 4, 1, 0),
        ]
        partner = [jnp.bitwise_xor(my, DIM_XOR[d]) for d in range(3)]

        barrier_sem = pltpu.get_barrier_semaphore()
        for d in range(3):
            pl.semaphore_signal(
                barrier_sem, inc=1,
                device_id=(partner[d],), device_id_type=pl.DeviceIdType.MESH,
            )
        pl.semaphore_wait(barrier_sem, 3)

        o = [part_starts[p] + jnp.int32(0) for p in range(3)]
        L = [PARTS[p] for p in range(3)]

        started = []
        for p in range(3):
            d = p
            half = L[p] // 2
            send_off = o[p] + (1 - bit[d]) * half
            base, slots = comm_bases[p]
            slot = base + slots[0]
            send_bf[pl.ds(slot, half), :] = jnp.dot(
                x_ref[pl.ds(send_off, half), :], w_ref[...],
                preferred_element_type=f32,
            ).astype(bf16)
            r = pltpu.make_async_remote_copy(
                src_ref=send_bf.at[pl.ds(slot, half), :],
                dst_ref=recv_bf.at[pl.ds(slot, half), :],
                send_sem=rs_sems.at[p, 0, 0],
                recv_sem=rs_sems.at[p, 1, 0],
                device_id=(partner[d],),
                device_id_type=pl.DeviceIdType.MESH,
            )
            r.start()
            started.append((p, d, half, slot, r))
        for p in range(3):
            half = L[p] // 2
            keep_off = o[p] + bit[p] * half
            out_ref[pl.ds(keep_off, half), :] = jnp.dot(
                x_ref[pl.ds(keep_off, half), :], w_ref[...],
                preferred_element_type=f32,
            )

        for k in range(3):
            if k > 0:
                started = []
                for p in range(3):
                    d = (p + k) % 3
                    half = L[p] // 2
                    base, slots = comm_bases[p]
                    slot = base + slots[k]
                    r = pltpu.make_async_remote_copy(
                        src_ref=send_bf.at[pl.ds(slot, half), :],
                        dst_ref=recv_bf.at[pl.ds(slot, half), :],
                        send_sem=rs_sems.at[p, 0, k],
                        recv_sem=rs_sems.at[p, 1, k],
                        device_id=(partner[d],),
                        device_id_type=pl.DeviceIdType.MESH,
                    )
                    r.start()
                    started.append((p, d, half, slot, r))
            ag_started = []
            for p, d, half, slot, r in started:
                r.wait()
                keep = o[p] + bit[d] * half
                o[p] = keep
                L[p] = half
                if k < 2:
                    nd = (p + k + 1) % 3
                    nh = half // 2
                    send_rel = (1 - bit[nd]) * nh
                    keep_rel = bit[nd] * nh
                    base, slots = comm_bases[p]
                    nslot = base + slots[k + 1]
                    send_bf[pl.ds(nslot, nh), :] = (
                        out_ref[pl.ds(keep + send_rel, nh), :]
                        + recv_bf[pl.ds(slot + send_rel, nh), :].astype(f32)
                    ).astype(bf16)
                    out_ref[pl.ds(keep + keep_rel, nh), :] += (
                        recv_bf[pl.ds(slot + keep_rel, nh), :].astype(f32)
                    )
                else:
                    out_ref[pl.ds(keep, half), :] = jnp.maximum(
                        out_ref[pl.ds(keep, half), :]
                        + recv_bf[pl.ds(slot, half), :].astype(f32),
                        0.0,
                    )
                    ag_bf[pl.ds(keep, half), :] = (
                        out_ref[pl.ds(keep, half), :].astype(bf16)
                    )
                    ag0 = pltpu.make_async_remote_copy(
                        src_ref=ag_bf.at[pl.ds(keep, half), :],
                        dst_ref=ag_bf.at[pl.ds(keep, half), :],
                        send_sem=ag_sems.at[p, 0, 0],
                        recv_sem=ag_sems.at[p, 1, 0],
                        device_id=(partner[d],),
                        device_id_type=pl.DeviceIdType.MESH,
                    )
                    ag0.start()
                    ag_started.append((p, d, ag0))

        for k in range(3):
            if k > 0:
                ag_started = []
                for p in range(3):
                    d = (p + 2 - k) % 3
                    rdma = pltpu.make_async_remote_copy(
                        src_ref=ag_bf.at[pl.ds(o[p], L[p]), :],
                        dst_ref=ag_bf.at[pl.ds(o[p], L[p]), :],
                        send_sem=ag_sems.at[p, 0, k],
                        recv_sem=ag_sems.at[p, 1, k],
                        device_id=(partner[d],),
                        device_id_type=pl.DeviceIdType.MESH,
                    )
                    rdma.start()
                    ag_started.append((p, d, rdma))
            for p, d, rdma in ag_started:
                rdma.wait()
                new_o = o[p] - bit[d] * L[p]
                p_off = new_o + (1 - bit[d]) * L[p]
                out_ref[pl.ds(p_off, L[p]), :] = (
                    ag_bf[pl.ds(p_off, L[p]), :].astype(f32)
                )
                o[p] = new_o
                L[p] = L[p] * 2

    return pl.pallas_call(
        body,
        out_shape=jax.ShapeDtypeStruct((m, n), f32),
        in_specs=[
            pl.BlockSpec(memory_space=pltpu.VMEM),
            pl.BlockSpec(memory_space=pltpu.VMEM),
        ],
        out_specs=pl.BlockSpec(memory_space=pltpu.VMEM),
        scratch_shapes=[
            pltpu.VMEM((comm_rows, n), bf16),
            pltpu.VMEM((comm_rows, n), bf16),
            pltpu.VMEM((m, n), bf16),
            pltpu.SemaphoreType.DMA((3, 2, 3)),
            pltpu.SemaphoreType.DMA((3, 2, 3)),
        ],
        compiler_params=pltpu.CompilerParams(collective_id=0),
    )(x, w_mat)


# device time: 84430 ns/iter; 4.2181x vs baseline; 1.0031x over previous
import jax
import jax.numpy as jnp
from jax import lax
from jax.experimental import pallas as pl
from jax.experimental.pallas import tpu as pltpu

N_DEV = 8
PARTS = (640, 640, 768)
DIM_XOR = (1, 3, 4)


def kernel(x, w_mat):
    m, _ = x.shape
    _, n = w_mat.shape
    assert sum(PARTS) == m

    part_starts = []
    comm_bases = []
    off = 0
    roff = 0
    for p_len in PARTS:
        part_starts.append(off)
        off += p_len
        comm_bases.append((roff, (0, p_len // 2, p_len // 2 + p_len // 4)))
        roff += p_len // 2 + p_len // 4 + p_len // 8
    comm_rows = roff

    f32 = jnp.float32
    bf16 = jnp.bfloat16

    def body(x_ref, w_ref, out_ref, recv_bf, send_bf, ag_bf,
             rs_sems, ag_sems):
        my = lax.axis_index("i")
        my4 = lax.rem(my, 4)
        bit = [
            jnp.where((my4 == 1) | (my4 == 2), 1, 0),
            jnp.where(my4 >= 2, 1, 0),
            jnp.where(my >= 4, 1, 0),
        ]
        partner = [jnp.bitwise_xor(my, DIM_XOR[d]) for d in range(3)]

        barrier_sem = pltpu.get_barrier_semaphore()
        for d in range(3):
            pl.semaphore_signal(
                barrier_sem, inc=1,
                device_id=(partner[d],), device_id_type=pl.DeviceIdType.MESH,
            )
        pl.semaphore_wait(barrier_sem, 3)

        o = [part_starts[p] + jnp.int32(0) for p in range(3)]
        L = [PARTS[p] for p in range(3)]

        started = []
        for p in range(3):
            d = p
            half = L[p] // 2
            send_off = o[p] + (1 - bit[d]) * half
            base, slots = comm_bases[p]
            slot = base + slots[0]
            send_bf[pl.ds(slot, half), :] = jnp.dot(
                x_ref[pl.ds(send_off, half), :], w_ref[...],
                preferred_element_type=f32,
            ).astype(bf16)
            r = pltpu.make_async_remote_copy(
                src_ref=send_bf.at[pl.ds(slot, half), :],
                dst_ref=recv_bf.at[pl.ds(slot, half), :],
                send_sem=rs_sems.at[p, 0, 0],
                recv_sem=rs_sems.at[p, 1, 0],
                device_id=(partner[d],),
                device_id_type=pl.DeviceIdType.MESH,
            )
            r.start()
            started.append((p, d, half, slot, r))
        for p in range(3):
            half = L[p] // 2
            keep_off = o[p] + bit[p] * half
            out_ref[pl.ds(keep_off, half), :] = jnp.dot(
                x_ref[pl.ds(keep_off, half), :], w_ref[...],
                preferred_element_type=f32,
            )

        for k in range(3):
            if k > 0:
                started = []
                for p in range(3):
                    d = (p + k) % 3
                    half = L[p] // 2
                    base, slots = comm_bases[p]
                    slot = base + slots[k]
                    r = pltpu.make_async_remote_copy(
                        src_ref=send_bf.at[pl.ds(slot, half), :],
                        dst_ref=recv_bf.at[pl.ds(slot, half), :],
                        send_sem=rs_sems.at[p, 0, k],
                        recv_sem=rs_sems.at[p, 1, k],
                        device_id=(partner[d],),
                        device_id_type=pl.DeviceIdType.MESH,
                    )
                    r.start()
                    started.append((p, d, half, slot, r))
            ag_started = []
            for p, d, half, slot, r in started:
                r.wait()
                keep = o[p] + bit[d] * half
                o[p] = keep
                L[p] = half
                if k < 2:
                    nd = (p + k + 1) % 3
                    nh = half // 2
                    send_rel = (1 - bit[nd]) * nh
                    keep_rel = bit[nd] * nh
                    base, slots = comm_bases[p]
                    nslot = base + slots[k + 1]
                    send_bf[pl.ds(nslot, nh), :] = (
                        out_ref[pl.ds(keep + send_rel, nh), :]
                        + recv_bf[pl.ds(slot + send_rel, nh), :].astype(f32)
                    ).astype(bf16)
                    out_ref[pl.ds(keep + keep_rel, nh), :] += (
                        recv_bf[pl.ds(slot + keep_rel, nh), :].astype(f32)
                    )
                else:
                    out_ref[pl.ds(keep, half), :] = jnp.maximum(
                        out_ref[pl.ds(keep, half), :]
                        + recv_bf[pl.ds(slot, half), :].astype(f32),
                        0.0,
                    )
                    ag_bf[pl.ds(keep, half), :] = (
                        out_ref[pl.ds(keep, half), :].astype(bf16)
                    )
                    ag0 = pltpu.make_async_remote_copy(
                        src_ref=ag_bf.at[pl.ds(keep, half), :],
                        dst_ref=ag_bf.at[pl.ds(keep, half), :],
                        send_sem=ag_sems.at[p, 0, 0, 0],
                        recv_sem=ag_sems.at[p, 1, 0, 0],
                        device_id=(partner[d],),
                        device_id_type=pl.DeviceIdType.MESH,
                    )
                    ag0.start()
                    ag_started.append((p, d, [ag0]))

        for k in range(3):
            if k > 0:
                ag_started = []
                for p in range(3):
                    d = (p + 2 - k) % 3
                    nsub = 2 if k == 2 else 1
                    sub = L[p] // nsub
                    rds = []
                    for j in range(nsub):
                        rdma = pltpu.make_async_remote_copy(
                            src_ref=ag_bf.at[pl.ds(o[p] + j * sub, sub), :],
                            dst_ref=ag_bf.at[pl.ds(o[p] + j * sub, sub), :],
                            send_sem=ag_sems.at[p, 0, k, j],
                            recv_sem=ag_sems.at[p, 1, k, j],
                            device_id=(partner[d],),
                            device_id_type=pl.DeviceIdType.MESH,
                        )
                        rdma.start()
                        rds.append(rdma)
                    ag_started.append((p, d, rds))
            for p, d, rds in ag_started:
                new_o = o[p] - bit[d] * L[p]
                p_off = new_o + (1 - bit[d]) * L[p]
                sub = L[p] // len(rds)
                for j, rdma in enumerate(rds):
                    rdma.wait()
                    out_ref[pl.ds(p_off + j * sub, sub), :] = (
                        ag_bf[pl.ds(p_off + j * sub, sub), :].astype(f32)
                    )
                o[p] = new_o
                L[p] = L[p] * 2

    return pl.pallas_call(
        body,
        out_shape=jax.ShapeDtypeStruct((m, n), f32),
        in_specs=[
            pl.BlockSpec(memory_space=pltpu.VMEM),
            pl.BlockSpec(memory_space=pltpu.VMEM),
        ],
        out_specs=pl.BlockSpec(memory_space=pltpu.VMEM),
        scratch_shapes=[
            pltpu.VMEM((comm_rows, n), bf16),
            pltpu.VMEM((comm_rows, n), bf16),
            pltpu.VMEM((m, n), bf16),
            pltpu.SemaphoreType.DMA((3, 2, 3)),
            pltpu.SemaphoreType.DMA((3, 2, 3, 2)),
        ],
        compiler_params=pltpu.CompilerParams(collective_id=0),
    )(x, w_mat)


# device time: 83659 ns/iter; 4.2570x vs baseline; 1.0092x over previous
import jax
import jax.numpy as jnp
from jax import lax
from jax.experimental import pallas as pl
from jax.experimental.pallas import tpu as pltpu

N_DEV = 8
PARTS = (640, 640, 768)
DIM_XOR = (1, 3, 4)


def kernel(x, w_mat):
    m, _ = x.shape
    _, n = w_mat.shape
    assert sum(PARTS) == m

    part_starts = []
    comm_bases = []
    off = 0
    roff = 0
    for p_len in PARTS:
        part_starts.append(off)
        off += p_len
        comm_bases.append((roff, (0, p_len // 2, p_len // 2 + p_len // 4)))
        roff += p_len // 2 + p_len // 4 + p_len // 8
    comm_rows = roff

    f32 = jnp.float32
    bf16 = jnp.bfloat16

    def body(x_ref, w_ref, out_ref, recv_bf, send_bf, ag_bf,
             rs_sems, ag_sems):
        my = lax.axis_index("i")
        my4 = lax.rem(my, 4)
        bit = [
            jnp.where((my4 == 1) | (my4 == 2), 1, 0),
            jnp.where(my4 >= 2, 1, 0),
            jnp.where(my >= 4, 1, 0),
        ]
        partner = [jnp.bitwise_xor(my, DIM_XOR[d]) for d in range(3)]

        barrier_sem = pltpu.get_barrier_semaphore()
        for d in range(3):
            pl.semaphore_signal(
                barrier_sem, inc=1,
                device_id=(partner[d],), device_id_type=pl.DeviceIdType.MESH,
            )
        pl.semaphore_wait(barrier_sem, 3)

        o = [part_starts[p] + jnp.int32(0) for p in range(3)]
        L = [PARTS[p] for p in range(3)]

        started = []
        for p in (2, 0, 1):
            d = p
            half = L[p] // 2
            send_off = o[p] + (1 - bit[d]) * half
            base, slots = comm_bases[p]
            slot = base + slots[0]
            send_bf[pl.ds(slot, half), :] = jnp.dot(
                x_ref[pl.ds(send_off, half), :], w_ref[...],
                preferred_element_type=f32,
            ).astype(bf16)
            r = pltpu.make_async_remote_copy(
                src_ref=send_bf.at[pl.ds(slot, half), :],
                dst_ref=recv_bf.at[pl.ds(slot, half), :],
                send_sem=rs_sems.at[p, 0, 0],
                recv_sem=rs_sems.at[p, 1, 0],
                device_id=(partner[d],),
                device_id_type=pl.DeviceIdType.MESH,
            )
            r.start()
            started.append((p, d, half, slot, r))
        for p in range(3):
            half = L[p] // 2
            keep_off = o[p] + bit[p] * half
            out_ref[pl.ds(keep_off, half), :] = jnp.dot(
                x_ref[pl.ds(keep_off, half), :], w_ref[...],
                preferred_element_type=f32,
            )

        for k in range(3):
            if k > 0:
                started = []
                for p in range(3):
                    d = (p + k) % 3
                    half = L[p] // 2
                    base, slots = comm_bases[p]
                    slot = base + slots[k]
                    r = pltpu.make_async_remote_copy(
                        src_ref=send_bf.at[pl.ds(slot, half), :],
                        dst_ref=recv_bf.at[pl.ds(slot, half), :],
                        send_sem=rs_sems.at[p, 0, k],
                        recv_sem=rs_sems.at[p, 1, k],
                        device_id=(partner[d],),
                        device_id_type=pl.DeviceIdType.MESH,
                    )
                    r.start()
                    started.append((p, d, half, slot, r))
            ag_started = []
            for p, d, half, slot, r in started:
                r.wait()
                keep = o[p] + bit[d] * half
                o[p] = keep
                L[p] = half
                if k < 2:
                    nd = (p + k + 1) % 3
                    nh = half // 2
                    send_rel = (1 - bit[nd]) * nh
                    keep_rel = bit[nd] * nh
                    base, slots = comm_bases[p]
                    nslot = base + slots[k + 1]
                    send_bf[pl.ds(nslot, nh), :] = (
                        out_ref[pl.ds(keep + send_rel, nh), :]
                        + recv_bf[pl.ds(slot + send_rel, nh), :].astype(f32)
                    ).astype(bf16)
                    out_ref[pl.ds(keep + keep_rel, nh), :] += (
                        recv_bf[pl.ds(slot + keep_rel, nh), :].astype(f32)
                    )
                else:
                    out_ref[pl.ds(keep, half), :] = jnp.maximum(
                        out_ref[pl.ds(keep, half), :]
                        + recv_bf[pl.ds(slot, half), :].astype(f32),
                        0.0,
                    )
                    ag_bf[pl.ds(keep, half), :] = (
                        out_ref[pl.ds(keep, half), :].astype(bf16)
                    )
                    ag0 = pltpu.make_async_remote_copy(
                        src_ref=ag_bf.at[pl.ds(keep, half), :],
                        dst_ref=ag_bf.at[pl.ds(keep, half), :],
                        send_sem=ag_sems.at[p, 0, 0, 0],
                        recv_sem=ag_sems.at[p, 1, 0, 0],
                        device_id=(partner[d],),
                        device_id_type=pl.DeviceIdType.MESH,
                    )
                    ag0.start()
                    ag_started.append((p, d, [ag0]))

        for k in range(3):
            if k > 0:
                ag_started = []
                for p in range(3):
                    d = (p + 2 - k) % 3
                    nsub = 2 if k == 2 else 1
                    sub = L[p] // nsub
                    rds = []
                    for j in range(nsub):
                        rdma = pltpu.make_async_remote_copy(
                            src_ref=ag_bf.at[pl.ds(o[p] + j * sub, sub), :],
                            dst_ref=ag_bf.at[pl.ds(o[p] + j * sub, sub), :],
                            send_sem=ag_sems.at[p, 0, k, j],
                            recv_sem=ag_sems.at[p, 1, k, j],
                            device_id=(partner[d],),
                            device_id_type=pl.DeviceIdType.MESH,
                        )
                        rdma.start()
                        rds.append(rdma)
                    ag_started.append((p, d, rds))
            for p, d, rds in ag_started:
                new_o = o[p] - bit[d] * L[p]
                p_off = new_o + (1 - bit[d]) * L[p]
                sub = L[p] // len(rds)
                for j, rdma in enumerate(rds):
                    rdma.wait()
                    out_ref[pl.ds(p_off + j * sub, sub), :] = (
                        ag_bf[pl.ds(p_off + j * sub, sub), :].astype(f32)
                    )
                o[p] = new_o
                L[p] = L[p] * 2

    return pl.pallas_call(
        body,
        out_shape=jax.ShapeDtypeStruct((m, n), f32),
        in_specs=[
            pl.BlockSpec(memory_space=pltpu.VMEM),
            pl.BlockSpec(memory_space=pltpu.VMEM),
        ],
        out_specs=pl.BlockSpec(memory_space=pltpu.VMEM),
        scratch_shapes=[
            pltpu.VMEM((comm_rows, n), bf16),
            pltpu.VMEM((comm_rows, n), bf16),
            pltpu.VMEM((m, n), bf16),
            pltpu.SemaphoreType.DMA((3, 2, 3)),
            pltpu.SemaphoreType.DMA((3, 2, 3, 2)),
        ],
        compiler_params=pltpu.CompilerParams(collective_id=0),
    )(x, w_mat)
